# Initial kernel scaffold; baseline (speedup 1.0000x reference)
import jax
import jax.numpy as jnp
from jax.experimental import pallas as pl


def kernel(x, conv_w, conv_b, gate_conv_w, gate_conv_b, ln1_g, ln1_b, ln2_g, ln2_b, attn_in_w, attn_gate_w, attn_gate_b):
    raise NotImplementedError("write your pallas kernel here")


# speedup vs baseline: 4.7289x; 4.7289x over previous
import math
from functools import partial

import jax
import jax.numpy as jnp
from jax.experimental import pallas as pl
from jax.experimental.pallas import tpu as pltpu

_LOG2E = 1.4426950408889634


def _elu(z):
    return jnp.where(z > 0, z, jnp.exp(jnp.minimum(z, 0.0)) - 1.0)


def _pos_enc(seq_len, num_channels):
    position = jnp.arange(seq_len, dtype=jnp.float32)
    num_timescales = num_channels // 2
    log_inc = math.log(10000.0) / (num_timescales - 1)
    inv_ts = jnp.exp(jnp.arange(num_timescales, dtype=jnp.float32) * -log_inc)
    scaled = position[:, None] * inv_ts[None, :]
    return jnp.concatenate([jnp.sin(scaled), jnp.cos(scaled)], axis=1)


def _block_kernel(x_ref, wc_ref, bc_ref, wg_ref, bg_ref, g1_ref, b1_ref,
                  pos_ref, wp_ref, wa_ref, ba_ref, g2_ref, b2_ref, o_ref,
                  *, H, W, C, num_heads, eps):
    Wp = W + 16
    Nout = H * Wp
    S = H * W
    d = C // num_heads

    xp = x_ref[0]
    ce = jnp.concatenate([_elu(xp), _elu(-xp)], axis=-1).astype(jnp.bfloat16)
    taps = []
    for kh in range(3):
        for kw in range(3):
            st = (kh + 1) * Wp + kw - 1
            taps.append(ce[st:st + Nout, :])
    wide = jnp.concatenate(taps, axis=-1)
    h = jnp.dot(wide, wc_ref[...], preferred_element_type=jnp.float32) + bc_ref[...]
    ge = jnp.concatenate([_elu(h), _elu(-h)], axis=-1).astype(jnp.bfloat16)
    g = jnp.dot(ge, wg_ref[...], preferred_element_type=jnp.float32) + bg_ref[...]
    y = g[:, :C] * jax.nn.sigmoid(g[:, C:]) + xp[2 * Wp:2 * Wp + Nout, :]
    mu = jnp.mean(y, axis=-1, keepdims=True)
    var = jnp.mean(jnp.square(y - mu), axis=-1, keepdims=True)
    x1f = (y - mu) * jax.lax.rsqrt(var + eps) * g1_ref[...] + b1_ref[...]
    x1 = x1f.reshape(H, Wp, C)[:, 8:8 + W, :].reshape(S, C)

    xa = (x1 + pos_ref[...]).astype(jnp.bfloat16)
    qkv = jnp.dot(xa, wp_ref[...], preferred_element_type=jnp.float32)
    k = qkv[:, :C].astype(jnp.bfloat16)
    v = qkv[:, C:2 * C].astype(jnp.bfloat16)
    q = (qkv[:, 2 * C:] * (d ** -0.5 * _LOG2E)).astype(jnp.bfloat16)
    outs = []
    for hh in range(num_heads):
        sl = slice(hh * d, (hh + 1) * d)
        s = jax.lax.dot_general(q[:, sl], k[:, sl], (((1,), (1,)), ((), ())),
                                preferred_element_type=jnp.float32)
        p = jnp.exp2(s - jnp.max(s, axis=-1, keepdims=True))
        l = jnp.sum(p, axis=-1, keepdims=True)
        o = jnp.dot(p.astype(jnp.bfloat16), v[:, sl],
                    preferred_element_type=jnp.float32)
        outs.append(o * pl.reciprocal(l, approx=True))
    attn = jnp.concatenate(outs, axis=-1).astype(jnp.bfloat16)
    g2 = jnp.dot(attn, wa_ref[...], preferred_element_type=jnp.float32) + ba_ref[...]
    y2 = g2[:, :C] * jax.nn.sigmoid(g2[:, C:]) + x1
    mu2 = jnp.mean(y2, axis=-1, keepdims=True)
    var2 = jnp.mean(jnp.square(y2 - mu2), axis=-1, keepdims=True)
    o_ref[0] = (y2 - mu2) * jax.lax.rsqrt(var2 + eps) * g2_ref[...] + b2_ref[...]


def kernel(x, conv_w, conv_b, gate_conv_w, gate_conv_b, ln1_g, ln1_b,
           ln2_g, ln2_b, attn_in_w, attn_gate_w, attn_gate_b):
    B, C, H, W = x.shape
    num_heads = 8
    eps = 1e-5
    Wp = W + 16
    Hp = H + 4
    S = H * W
    x_nhwc = jnp.transpose(x, (0, 2, 3, 1))
    xpad = jnp.pad(x_nhwc, ((0, 0), (2, 2), (8, 8), (0, 0))).reshape(B, Hp * Wp, C)
    wc = conv_w.reshape(9 * 2 * C, C).astype(jnp.bfloat16)
    wg = gate_conv_w.astype(jnp.bfloat16)
    wp = attn_in_w.astype(jnp.bfloat16)
    wa = attn_gate_w.astype(jnp.bfloat16)
    pos = _pos_enc(S, C)
    out = pl.pallas_call(
        partial(_block_kernel, H=H, W=W, C=C, num_heads=num_heads, eps=eps),
        out_shape=jax.ShapeDtypeStruct((B, S, C), jnp.float32),
        grid=(B,),
        in_specs=[
            pl.BlockSpec((1, Hp * Wp, C), lambda b: (b, 0, 0)),
            pl.BlockSpec((9 * 2 * C, C), lambda b: (0, 0)),
            pl.BlockSpec((1, C), lambda b: (0, 0)),
            pl.BlockSpec((2 * C, 2 * C), lambda b: (0, 0)),
            pl.BlockSpec((1, 2 * C), lambda b: (0, 0)),
            pl.BlockSpec((1, C), lambda b: (0, 0)),
            pl.BlockSpec((1, C), lambda b: (0, 0)),
            pl.BlockSpec((S, C), lambda b: (0, 0)),
            pl.BlockSpec((C, 3 * C), lambda b: (0, 0)),
            pl.BlockSpec((C, 2 * C), lambda b: (0, 0)),
            pl.BlockSpec((1, 2 * C), lambda b: (0, 0)),
            pl.BlockSpec((1, C), lambda b: (0, 0)),
            pl.BlockSpec((1, C), lambda b: (0, 0)),
        ],
        out_specs=pl.BlockSpec((1, S, C), lambda b: (b, 0, 0)),
        compiler_params=pltpu.CompilerParams(
            dimension_semantics=("parallel",),
            vmem_limit_bytes=100 * 1024 * 1024,
        ),
    )(xpad, wc, conv_b.reshape(1, C), wg, gate_conv_b.reshape(1, 2 * C),
      ln1_g.reshape(1, C), ln1_b.reshape(1, C),
      pos, wp, wa, attn_gate_b.reshape(1, 2 * C),
      ln2_g.reshape(1, C), ln2_b.reshape(1, C))
    return jnp.transpose(out.reshape(B, H, W, C), (0, 3, 1, 2))
